# Initial kernel scaffold; baseline (speedup 1.0000x reference)
import jax
import jax.numpy as jnp
from jax.experimental import pallas as pl


def kernel(x, pos_keypoints, keypoints):
    raise NotImplementedError("write your pallas kernel here")


# speedup vs baseline: 14.1854x; 14.1854x over previous
import jax
import jax.numpy as jnp
from jax import lax
from jax.experimental import pallas as pl

B, C, H, W = 4, 384, 14, 14
IMG = 224
SCALE = IMG // H
P = 50
N = 50
ALPHA = 0.7
K = C // 2
HW = H * W


def _tt_kernel(x_ref, posk_ref, negk_ref, out_ref, fm_ref):
    xf = x_ref[0]
    pos = posk_ref[0]
    neg = negk_ref[0]

    idx_p = (pos[:, 0:1] // SCALE) * W + pos[:, 1:2] // SCALE
    idx_n = (neg[:, 0:1] // SCALE) * W + neg[:, 1:2] // SCALE

    pix_p = lax.broadcasted_iota(jnp.int32, (P, HW), 1)
    pix_n = lax.broadcasted_iota(jnp.int32, (N, HW), 1)
    cnt_p = jnp.sum((idx_p == pix_p).astype(jnp.float32), axis=0, keepdims=True)
    cnt_n = jnp.sum((idx_n == pix_n).astype(jnp.float32), axis=0, keepdims=True)
    w = ALPHA * cnt_p - (1.0 - ALPHA) * cnt_n

    scores = jnp.sum(xf * w, axis=1, keepdims=True)

    ii = lax.broadcasted_iota(jnp.int32, (C, C), 0)
    jj = lax.broadcasted_iota(jnp.int32, (C, C), 1)
    s_row = jnp.sum(jnp.where(ii == jj, scores, 0.0), axis=0, keepdims=True)

    before = (s_row < scores) | ((s_row == scores) & (jj < ii))
    rank = jnp.sum(before.astype(jnp.int32), axis=1, keepdims=True)

    keep = (rank >= K).astype(jnp.float32)
    out_ref[0] = xf * keep

    rr = lax.broadcasted_iota(jnp.int32, (C, K), 1)
    chan = lax.broadcasted_iota(jnp.int32, (C, K), 0)
    hits = jnp.where(rank == rr, chan, 0)
    fm_ref[0] = jnp.sum(hits, axis=0, keepdims=True)


@jax.jit
def kernel(x, pos_keypoints, keypoints):
    xf = x.reshape(B, C, HW)
    out_flat, fm = pl.pallas_call(
        _tt_kernel,
        grid=(B,),
        in_specs=[
            pl.BlockSpec((1, C, HW), lambda b: (b, 0, 0)),
            pl.BlockSpec((1, P, 2), lambda b: (b, 0, 0)),
            pl.BlockSpec((1, N, 2), lambda b: (b, 0, 0)),
        ],
        out_specs=(
            pl.BlockSpec((1, C, HW), lambda b: (b, 0, 0)),
            pl.BlockSpec((1, 1, K), lambda b: (b, 0, 0)),
        ),
        out_shape=(
            jax.ShapeDtypeStruct((B, C, HW), jnp.float32),
            jax.ShapeDtypeStruct((B, 1, K), jnp.int32),
        ),
    )(xf, pos_keypoints, keypoints)
    return out_flat.reshape(B, C, H, W), fm.reshape(B, K)
